# baseline (speedup 1.0000x reference)
import numpy as np

import jax
import jax.numpy as jnp
from jax import lax
from jax.experimental import pallas as pl
from jax.experimental.pallas import tpu as pltpu

_NEG_SLOPE = 1.0 / 20.0
_HI = lax.Precision.HIGHEST


def _shl(x, d):
    return jnp.concatenate([x[:, d:], x[:, :d]], axis=1)


def _cnn_block_kernel(x_ref, bw1_ref, w2b_ref, w3b_ref, aux_ref, o_ref):
    bt = x_ref.shape[-1]
    bw1 = bw1_ref[...]
    ps = []
    for pr in range(5):
        m = None
        for dr in range(8):
            r = pr * 8 + dr
            slab = x_ref[:, r:r + 4, :, :].reshape(576, bt)
            y = lax.dot_general(slab, bw1, (((0,), (0,)), ((), ())),
                                preferred_element_type=jnp.float32)
            m = y if m is None else jnp.maximum(m, y)
        for d in (1, 2, 4):
            m = jnp.maximum(m, _shl(m, d))
        z = m + aux_ref[0:1, :]
        ps.append(jnp.maximum(z, z * _NEG_SLOPE).astype(jnp.bfloat16))

    p2 = jnp.concatenate(ps, axis=1)
    y2 = lax.dot_general(p2, w2b_ref[...], (((1,), (0,)), ((), ())),
                         preferred_element_type=jnp.float32)
    for d in (1, 2, 4, 8):
        y2 = jnp.maximum(y2, _shl(y2, d))
    z2 = y2 + aux_ref[1:2, 0:256]
    f = jnp.maximum(z2, z2 * _NEG_SLOPE)

    out = lax.dot_general(f.astype(jnp.bfloat16), w3b_ref[...],
                          (((1,), (0,)), ((), ())),
                          preferred_element_type=jnp.float32)
    o_ref[...] = out + aux_ref[2:3, 0:128]


def _conv1_struct():
    s = np.zeros((4, 48, 48), np.float32)
    for kw in range(4):
        for c in range(48 - kw):
            s[kw, c + kw, c] = 1.0
    return s


def _conv2_struct():
    s = np.zeros((5, 48, 4, 4, 2, 2), np.float32)
    for r2 in range(4):
        for c2 in range(4):
            for kh in range(2):
                for kw in range(2):
                    s[r2 + kh, 8 * (c2 + kw), r2, c2, kh, kw] = 1.0
    return s


_S1 = _conv1_struct()
_S2 = _conv2_struct()


def kernel(x, w1, b1, w2, b2, w3, b3):
    B = x.shape[0]
    assert x.shape[1:] == (3, 48, 48), x.shape
    BT = 128
    g = pl.cdiv(B, BT)
    Bp = g * BT

    xt = jnp.transpose(x, (1, 2, 3, 0)).astype(jnp.bfloat16)

    bw1 = jnp.einsum('kwc,oihk->ihwoc', _S1, w1).reshape(576, 384)
    bw1 = bw1.astype(jnp.bfloat16)
    w2b = jnp.einsum('pqrshw,oihw->piqors', _S2, w2).reshape(1920, 256)
    w2b = w2b.astype(jnp.bfloat16)
    w3p = jnp.pad(jnp.transpose(w3, (1, 0)), ((0, 0), (0, 122)))
    w3b = jnp.concatenate([w3p[:, None, :],
                           jnp.zeros((16, 15, 128), jnp.float32)],
                          axis=1).reshape(256, 128).astype(jnp.bfloat16)

    aux = jnp.zeros((8, 384), jnp.float32)
    aux = aux.at[0, :].set(jnp.repeat(b1.astype(jnp.float32), 48))
    aux = aux.at[1, 0:256].set(jnp.repeat(b2.astype(jnp.float32), 16))
    aux = aux.at[2, 0:6].set(b3.astype(jnp.float32))

    out = pl.pallas_call(
        _cnn_block_kernel,
        out_shape=jax.ShapeDtypeStruct((g, BT, 128), jnp.float32),
        grid=(g,),
        in_specs=[
            pl.BlockSpec((3, 48, 48, BT), lambda i: (0, 0, 0, i)),
            pl.BlockSpec((576, 384), lambda i: (0, 0)),
            pl.BlockSpec((1920, 256), lambda i: (0, 0)),
            pl.BlockSpec((256, 128), lambda i: (0, 0)),
            pl.BlockSpec((8, 384), lambda i: (0, 0)),
        ],
        out_specs=pl.BlockSpec((None, BT, 128), lambda i: (i, 0, 0)),
        compiler_params=pltpu.CompilerParams(
            dimension_semantics=("arbitrary",),
            vmem_limit_bytes=48 * 1024 * 1024,
        ),
    )(xt, bw1, w2b, w3b, aux)
    return out.reshape(Bp, 128)[:B, :6]


# speedup vs baseline: 1.2210x; 1.2210x over previous
import numpy as np

import jax
import jax.numpy as jnp
from jax import lax
from jax.experimental import pallas as pl
from jax.experimental.pallas import tpu as pltpu

_NEG_SLOPE = 1.0 / 20.0
_HI = lax.Precision.HIGHEST


def _shl(x, d):
    return jnp.concatenate([x[:, d:], x[:, :d]], axis=1)


def _cnn_block_kernel(x_ref, bwa_ref, bwb_ref, w2b_ref, w3b_ref, aux_ref, o_ref):
    bt = x_ref.shape[-1]
    bwa = bwa_ref[...]
    bwb = bwb_ref[...]
    ps = []
    for pr in range(5):
        m = None
        for dr in range(8):
            r = pr * 8 + dr
            sa = x_ref[:, r:r + 4, 0:40, :].reshape(480, bt)
            sb = x_ref[:, r:r + 4, 32:48, :].reshape(192, bt)
            ya = lax.dot_general(sa, bwa, (((0,), (0,)), ((), ())),
                                 preferred_element_type=jnp.float32)
            yb = lax.dot_general(sb, bwb, (((0,), (0,)), ((), ())),
                                 preferred_element_type=jnp.float32)
            y = jnp.concatenate([ya, yb], axis=1)
            m = y if m is None else jnp.maximum(m, y)
        for d in (1, 2, 4):
            m = jnp.maximum(m, _shl(m, d))
        z = m + aux_ref[0:1, :]
        ps.append(jnp.maximum(z, z * _NEG_SLOPE).astype(jnp.bfloat16))

    p2 = jnp.concatenate(ps, axis=1)
    y2 = lax.dot_general(p2, w2b_ref[...], (((1,), (0,)), ((), ())),
                         preferred_element_type=jnp.float32)
    for d in (1, 2, 4, 8):
        y2 = jnp.maximum(y2, _shl(y2, d))
    z2 = y2 + aux_ref[1:2, 0:256]
    f = jnp.maximum(z2, z2 * _NEG_SLOPE)

    out = lax.dot_general(f.astype(jnp.bfloat16), w3b_ref[...],
                          (((1,), (0,)), ((), ())),
                          preferred_element_type=jnp.float32)
    o_ref[...] = out + aux_ref[2:3, 0:128]


def _conv1_struct_a():
    s = np.zeros((4, 40, 32), np.float32)
    for kw in range(4):
        for c in range(32):
            s[kw, c + kw, c] = 1.0
    return s


def _conv1_struct_b():
    s = np.zeros((4, 16, 16), np.float32)
    for kw in range(4):
        for cb in range(8):
            s[kw, cb + kw, cb] = 1.0
    return s


def _lane_of_pool(ci, pc):
    return ci * 32 + 8 * pc if pc < 4 else 256 + ci * 16


def _conv2_struct():
    s = np.zeros((5, 384, 8, 4, 4, 2, 2), np.float32)
    for ci in range(8):
        for r2 in range(4):
            for c2 in range(4):
                for kh in range(2):
                    for kw in range(2):
                        s[r2 + kh, _lane_of_pool(ci, c2 + kw), ci,
                          r2, c2, kh, kw] = 1.0
    return s


_S1A = _conv1_struct_a()
_S1B = _conv1_struct_b()
_S2 = _conv2_struct()


def kernel(x, w1, b1, w2, b2, w3, b3):
    B = x.shape[0]
    assert x.shape[1:] == (3, 48, 48), x.shape
    BT = 128
    g = pl.cdiv(B, BT)
    Bp = g * BT

    xt = jnp.transpose(x, (1, 2, 3, 0)).astype(jnp.bfloat16)

    bwa = jnp.einsum('kwc,oihk->ihwoc', _S1A, w1).reshape(480, 256)
    bwa = bwa.astype(jnp.bfloat16)
    bwb = jnp.einsum('kwc,oihk->ihwoc', _S1B, w1).reshape(192, 128)
    bwb = bwb.astype(jnp.bfloat16)
    w2b = jnp.einsum('pqirshw,oihw->pqors', _S2, w2).reshape(1920, 256)
    w2b = w2b.astype(jnp.bfloat16)
    w3p = jnp.pad(jnp.transpose(w3, (1, 0)), ((0, 0), (0, 122)))
    w3b = jnp.concatenate([w3p[:, None, :],
                           jnp.zeros((16, 15, 128), jnp.float32)],
                          axis=1).reshape(256, 128).astype(jnp.bfloat16)

    aux = jnp.zeros((8, 384), jnp.float32)
    b1f = b1.astype(jnp.float32)
    aux = aux.at[0, :].set(jnp.concatenate([jnp.repeat(b1f, 32),
                                            jnp.repeat(b1f, 16)]))
    aux = aux.at[1, 0:256].set(jnp.repeat(b2.astype(jnp.float32), 16))
    aux = aux.at[2, 0:6].set(b3.astype(jnp.float32))

    out = pl.pallas_call(
        _cnn_block_kernel,
        out_shape=jax.ShapeDtypeStruct((g, BT, 128), jnp.float32),
        grid=(g,),
        in_specs=[
            pl.BlockSpec((3, 48, 48, BT), lambda i: (0, 0, 0, i)),
            pl.BlockSpec((480, 256), lambda i: (0, 0)),
            pl.BlockSpec((192, 128), lambda i: (0, 0)),
            pl.BlockSpec((1920, 256), lambda i: (0, 0)),
            pl.BlockSpec((256, 128), lambda i: (0, 0)),
            pl.BlockSpec((8, 384), lambda i: (0, 0)),
        ],
        out_specs=pl.BlockSpec((None, BT, 128), lambda i: (i, 0, 0)),
        compiler_params=pltpu.CompilerParams(
            dimension_semantics=("parallel",),
            vmem_limit_bytes=48 * 1024 * 1024,
        ),
    )(xt, bwa, bwb, w2b, w3b, aux)
    return out.reshape(Bp, 128)[:B, :6]
